# baseline (speedup 1.0000x reference)
import functools

import jax
import jax.numpy as jnp
from jax.experimental import pallas as pl
from jax.experimental.pallas import tpu as pltpu

NEIGH_HALF = 1.0
TP = 32
CC = 16


def _pool_kernel(epx_ref, epy_ref, apx_ref, apy_ref, lt_ref, h_ref, W_sp_ref,
                 b_sp_ref, W1_ref, b1_ref, W2_ref, b2_ref, out_ref, acc_ref,
                 *, nc, ncp2):
    tp = h_ref.shape[0]
    e64 = W1_ref.shape[0] - h_ref.shape[1]
    W1_top = W1_ref[:e64, :]
    A = jnp.dot(W_sp_ref[...], W1_top, preferred_element_type=jnp.float32)
    base = (jnp.dot(h_ref[...], W1_ref[e64:, :],
                    preferred_element_type=jnp.float32)
            + jnp.dot(b_sp_ref[...], W1_top,
                      preferred_element_type=jnp.float32)
            + b1_ref[...])

    rx = apx_ref[...] - epx_ref[...]
    ry = apy_ref[...] - epy_ref[...]
    rx = jnp.where(jnp.abs(rx) > NEIGH_HALF, 0.0, rx)
    ry = jnp.where(jnp.abs(ry) > NEIGH_HALF, 0.0, ry)

    active = jnp.where((rx != 0.0) | (ry != 0.0), 1.0, 0.0)
    nact = jnp.sum(active, axis=1, keepdims=True)
    anyzero = 1.0 - jnp.min(active, axis=1, keepdims=True)
    k = nact + anyzero
    km = jnp.max(k)
    pos = anyzero + jnp.dot(active, lt_ref[...],
                            preferred_element_type=jnp.float32)
    pos_i = pos.astype(jnp.int32)
    ii = jax.lax.broadcasted_iota(jnp.int32, (tp, nc, ncp2), 2)
    hit = (pos_i[:, :, None] == ii) & (active[:, :, None] != 0.0)
    relcx = jnp.sum(jnp.where(hit, rx[:, :, None], 0.0), axis=1)
    relcy = jnp.sum(jnp.where(hit, ry[:, :, None], 0.0), axis=1)
    lane = jax.lax.broadcasted_iota(jnp.int32, (tp, ncp2), 1)
    valid = lane < k.astype(jnp.int32)
    relcx = jnp.where(valid, relcx, relcx[:, 0:1])
    relcy = jnp.where(valid, relcy, relcy[:, 0:1])

    def chunk(j):
        rxc = relcx[:, j * CC:(j + 1) * CC]
        ryc = relcy[:, j * CC:(j + 1) * CC]
        pre1 = (ryc[:, :, None] * A[1][None, None, :]
                + (rxc[:, :, None] * A[0][None, None, :]
                   + base[:, None, :]))
        x1 = jnp.maximum(pre1, 0.0).reshape(tp * CC, -1)
        y = jnp.dot(x1, W2_ref[...], preferred_element_type=jnp.float32)
        return jnp.max(y.reshape(tp, CC, -1), axis=1)

    acc_ref[...] = chunk(0)
    for j in range(1, ncp2 // CC):
        @pl.when(km > float(j * CC))
        def _():
            acc_ref[...] = jnp.maximum(acc_ref[...], chunk(j))

    out_ref[...] = jnp.maximum(acc_ref[...] + b2_ref[...], 0.0)


def kernel(h_states, end_pos, rel_pos, annotated_points, W_sp, b_sp, W1, b1,
           W2, b2, seq_start_end):
    del rel_pos, seq_start_end
    h = h_states.reshape(-1, h_states.shape[-1])
    B = h.shape[0]
    NC = annotated_points.shape[0]
    BN = W2.shape[1]

    epx = end_pos[:, 0:1]
    epy = end_pos[:, 1:2]
    NCP = -(-NC // 8) * 8
    apx = annotated_points[:, 0].reshape(1, NC)
    apy = annotated_points[:, 1].reshape(1, NC)
    apx = jnp.concatenate([apx, jnp.broadcast_to(apx[:, :1], (1, NCP - NC))],
                          axis=1)
    apy = jnp.concatenate([apy, jnp.broadcast_to(apy[:, :1], (1, NCP - NC))],
                          axis=1)
    NCP2 = -(-(NCP + 1) // CC) * CC
    lt = (jax.lax.broadcasted_iota(jnp.int32, (NCP, NCP), 0)
          < jax.lax.broadcasted_iota(jnp.int32, (NCP, NCP), 1)
          ).astype(jnp.float32)

    full = lambda shape: pl.BlockSpec(shape, lambda i: (0, 0))
    out = pl.pallas_call(
        functools.partial(_pool_kernel, nc=NCP, ncp2=NCP2),
        grid=(B // TP,),
        in_specs=[
            pl.BlockSpec((TP, 1), lambda i: (i, 0)),
            pl.BlockSpec((TP, 1), lambda i: (i, 0)),
            full((1, NCP)),
            full((1, NCP)),
            full((NCP, NCP)),
            pl.BlockSpec((TP, h.shape[1]), lambda i: (i, 0)),
            full(W_sp.shape),
            full((1, b_sp.shape[0])),
            full(W1.shape),
            full((1, b1.shape[0])),
            full(W2.shape),
            full((1, b2.shape[0])),
        ],
        out_specs=pl.BlockSpec((TP, BN), lambda i: (i, 0)),
        out_shape=jax.ShapeDtypeStruct((B, BN), jnp.float32),
        scratch_shapes=[pltpu.VMEM((TP, BN), jnp.float32)],
    )(epx, epy, apx, apy, lt, h, W_sp, b_sp.reshape(1, -1), W1,
      b1.reshape(1, -1), W2, b2.reshape(1, -1))
    return out


# speedup vs baseline: 1.0967x; 1.0967x over previous
import functools

import jax
import jax.numpy as jnp
from jax.experimental import pallas as pl
from jax.experimental.pallas import tpu as pltpu

NEIGH_HALF = 1.0
TP = 128
CC = 32


def _pool_kernel(epx_ref, epy_ref, apx_ref, apy_ref, lt_ref, h_ref, W_sp_ref,
                 b_sp_ref, W1_ref, b1_ref, W2_ref, b2_ref, out_ref, acc_ref,
                 *, nc, ncp2):
    tp = h_ref.shape[0]
    e64 = W1_ref.shape[0] - h_ref.shape[1]
    W1_top = W1_ref[:e64, :]
    A = jnp.dot(W_sp_ref[...], W1_top, preferred_element_type=jnp.float32)
    base = (jnp.dot(h_ref[...], W1_ref[e64:, :],
                    preferred_element_type=jnp.float32)
            + jnp.dot(b_sp_ref[...], W1_top,
                      preferred_element_type=jnp.float32)
            + b1_ref[...])

    rx = apx_ref[...] - epx_ref[...]
    ry = apy_ref[...] - epy_ref[...]
    rx = jnp.where(jnp.abs(rx) > NEIGH_HALF, 0.0, rx)
    ry = jnp.where(jnp.abs(ry) > NEIGH_HALF, 0.0, ry)

    active = jnp.where((rx != 0.0) | (ry != 0.0), 1.0, 0.0)
    nact = jnp.sum(active, axis=1, keepdims=True)
    anyzero = 1.0 - jnp.min(active, axis=1, keepdims=True)
    k = nact + anyzero
    km = jnp.max(k)
    pos = anyzero + jnp.dot(active, lt_ref[...],
                            preferred_element_type=jnp.float32)
    pos_i = pos.astype(jnp.int32)
    ii = jax.lax.broadcasted_iota(jnp.int32, (tp, nc, ncp2), 2)
    hit = (pos_i[:, :, None] == ii) & (active[:, :, None] != 0.0)
    relcx = jnp.sum(jnp.where(hit, rx[:, :, None], 0.0), axis=1)
    relcy = jnp.sum(jnp.where(hit, ry[:, :, None], 0.0), axis=1)
    lane = jax.lax.broadcasted_iota(jnp.int32, (tp, ncp2), 1)
    valid = lane < k.astype(jnp.int32)
    relcx = jnp.where(valid, relcx, relcx[:, 0:1])
    relcy = jnp.where(valid, relcy, relcy[:, 0:1])

    def chunk(j):
        rxc = relcx[:, j * CC:(j + 1) * CC]
        ryc = relcy[:, j * CC:(j + 1) * CC]
        pre1 = (ryc[:, :, None] * A[1][None, None, :]
                + (rxc[:, :, None] * A[0][None, None, :]
                   + base[:, None, :]))
        x1 = jnp.maximum(pre1, 0.0).reshape(tp * CC, -1)
        y = jnp.dot(x1, W2_ref[...], preferred_element_type=jnp.float32)
        return jnp.max(y.reshape(tp, CC, -1), axis=1)

    acc_ref[...] = chunk(0)
    for j in range(1, ncp2 // CC):
        @pl.when(km > float(j * CC))
        def _():
            acc_ref[...] = jnp.maximum(acc_ref[...], chunk(j))

    out_ref[...] = jnp.maximum(acc_ref[...] + b2_ref[...], 0.0)


def kernel(h_states, end_pos, rel_pos, annotated_points, W_sp, b_sp, W1, b1,
           W2, b2, seq_start_end):
    del rel_pos, seq_start_end
    h = h_states.reshape(-1, h_states.shape[-1])
    B = h.shape[0]
    NC = annotated_points.shape[0]
    BN = W2.shape[1]

    epx = end_pos[:, 0:1]
    epy = end_pos[:, 1:2]
    NCP = -(-NC // 8) * 8
    apx = annotated_points[:, 0].reshape(1, NC)
    apy = annotated_points[:, 1].reshape(1, NC)
    apx = jnp.concatenate([apx, jnp.broadcast_to(apx[:, :1], (1, NCP - NC))],
                          axis=1)
    apy = jnp.concatenate([apy, jnp.broadcast_to(apy[:, :1], (1, NCP - NC))],
                          axis=1)
    NCP2 = -(-(NCP + 1) // CC) * CC
    lt = (jax.lax.broadcasted_iota(jnp.int32, (NCP, NCP), 0)
          < jax.lax.broadcasted_iota(jnp.int32, (NCP, NCP), 1)
          ).astype(jnp.float32)

    full = lambda shape: pl.BlockSpec(shape, lambda i: (0, 0))
    out = pl.pallas_call(
        functools.partial(_pool_kernel, nc=NCP, ncp2=NCP2),
        grid=(B // TP,),
        in_specs=[
            pl.BlockSpec((TP, 1), lambda i: (i, 0)),
            pl.BlockSpec((TP, 1), lambda i: (i, 0)),
            full((1, NCP)),
            full((1, NCP)),
            full((NCP, NCP)),
            pl.BlockSpec((TP, h.shape[1]), lambda i: (i, 0)),
            full(W_sp.shape),
            full((1, b_sp.shape[0])),
            full(W1.shape),
            full((1, b1.shape[0])),
            full(W2.shape),
            full((1, b2.shape[0])),
        ],
        out_specs=pl.BlockSpec((TP, BN), lambda i: (i, 0)),
        out_shape=jax.ShapeDtypeStruct((B, BN), jnp.float32),
        scratch_shapes=[pltpu.VMEM((TP, BN), jnp.float32)],
    )(epx, epy, apx, apy, lt, h, W_sp, b_sp.reshape(1, -1), W1,
      b1.reshape(1, -1), W2, b2.reshape(1, -1))
    return out


# speedup vs baseline: 1.2688x; 1.1569x over previous
import functools

import jax
import jax.numpy as jnp
from jax.experimental import pallas as pl

NEIGH_HALF = 1.0
TP = 128


def _pool_kernel(epx_ref, epy_ref, apx_ref, apy_ref, h_ref, W_sp_ref,
                 b_sp_ref, W1_ref, b1_ref, W2_ref, b2_ref, out_ref, *, nc):
    tp = h_ref.shape[0]
    e64 = W1_ref.shape[0] - h_ref.shape[1]
    W1_top = W1_ref[:e64, :]
    A = jnp.dot(W_sp_ref[...], W1_top, preferred_element_type=jnp.float32)
    base = (jnp.dot(h_ref[...], W1_ref[e64:, :],
                    preferred_element_type=jnp.float32)
            + jnp.dot(b_sp_ref[...], W1_top,
                      preferred_element_type=jnp.float32)
            + b1_ref[...])

    rx = apx_ref[...] - epx_ref[...]
    ry = apy_ref[...] - epy_ref[...]
    rx = jnp.where(jnp.abs(rx) > NEIGH_HALF, 0.0, rx)
    ry = jnp.where(jnp.abs(ry) > NEIGH_HALF, 0.0, ry)

    pre1 = (ry[:, :, None] * A[1][None, None, :]
            + (rx[:, :, None] * A[0][None, None, :]
               + base[:, None, :]))
    x1 = jnp.maximum(pre1, 0.0).reshape(tp * nc, -1)

    bn = W2_ref.shape[1]
    for j in range(bn // 512):
        y = jnp.dot(x1, W2_ref[:, j * 512:(j + 1) * 512],
                    preferred_element_type=jnp.float32)
        ymax = jnp.max(y.reshape(tp, nc, 512), axis=1)
        out_ref[:, j * 512:(j + 1) * 512] = jnp.maximum(
            ymax + b2_ref[:, j * 512:(j + 1) * 512], 0.0)


def kernel(h_states, end_pos, rel_pos, annotated_points, W_sp, b_sp, W1, b1,
           W2, b2, seq_start_end):
    del rel_pos, seq_start_end
    h = h_states.reshape(-1, h_states.shape[-1])
    B = h.shape[0]
    NC = annotated_points.shape[0]
    BN = W2.shape[1]

    epx = end_pos[:, 0:1]
    epy = end_pos[:, 1:2]
    NCP = -(-NC // 8) * 8
    apx = annotated_points[:, 0].reshape(1, NC)
    apy = annotated_points[:, 1].reshape(1, NC)
    apx = jnp.concatenate([apx, jnp.broadcast_to(apx[:, :1], (1, NCP - NC))],
                          axis=1)
    apy = jnp.concatenate([apy, jnp.broadcast_to(apy[:, :1], (1, NCP - NC))],
                          axis=1)
    NC = NCP

    full = lambda shape: pl.BlockSpec(shape, lambda i: (0, 0))
    out = pl.pallas_call(
        functools.partial(_pool_kernel, nc=NC),
        grid=(B // TP,),
        in_specs=[
            pl.BlockSpec((TP, 1), lambda i: (i, 0)),
            pl.BlockSpec((TP, 1), lambda i: (i, 0)),
            full((1, NC)),
            full((1, NC)),
            pl.BlockSpec((TP, h.shape[1]), lambda i: (i, 0)),
            full(W_sp.shape),
            full((1, b_sp.shape[0])),
            full(W1.shape),
            full((1, b1.shape[0])),
            full(W2.shape),
            full((1, b2.shape[0])),
        ],
        out_specs=pl.BlockSpec((TP, BN), lambda i: (i, 0)),
        out_shape=jax.ShapeDtypeStruct((B, BN), jnp.float32),
    )(epx, epy, apx, apy, h, W_sp, b_sp.reshape(1, -1), W1, b1.reshape(1, -1),
      W2, b2.reshape(1, -1))
    return out
